# baseline (speedup 1.0000x reference)
import functools

import jax
import jax.numpy as jnp
from jax import lax
from jax.experimental import pallas as pl
from jax.experimental.pallas import tpu as pltpu
from jax.experimental.pallas import tpu_sc as plsc

B, M, L, D = 4, 4096, 8192, 1024
NC, NS, LANES = 2, 16, 16
NW = NC * NS
TPB = NW // B
MPT = M // TPB
SUB = 32
NSUB = MPT // SUB
CAP = L + LANES
CBITS = 13
ROWB = SUB * D * 4
SCATB = LANES * D * 4


def _unmerge_body(merged_hbm, idx_hbm, out_hbm,
                  idxb_v, scale_v, stage_v, work_v, slab_v, sbuf_v, pidx_v,
                  ssem0, ssem1, osem0, osem1):
    ssems = (ssem0, ssem1)
    osems = (osem0, osem1)
    cid = lax.axis_index("c")
    sid = lax.axis_index("s")
    wid = sid * NC + cid
    b = wid // TPB
    mlo = (wid % TPB) * MPT
    sbase = wid * MPT

    iota = lax.iota(jnp.int32, LANES)

    pltpu.sync_copy(idx_hbm.at[b], idxb_v)

    def start_slab(r, slot):
        pltpu.async_copy(merged_hbm.at[pl.ds(sbase + r * SUB, SUB)],
                         slab_v.at[slot], ssems[slot])

    def wait_slab(r, slot):
        pltpu.make_async_copy(merged_hbm.at[pl.ds(sbase + r * SUB, SUB)],
                              slab_v.at[slot], ssems[slot]).wait()

    start_slab(0, 0)

    zeros = jnp.zeros((LANES,), jnp.float32)

    def zero_body(i, _):
        scale_v[pl.ds(i * LANES, LANES)] = zeros
        return 0

    lax.fori_loop(0, (MPT + LANES) // LANES, zero_body, 0)

    ones = jnp.ones((LANES,), jnp.float32)

    def hist_body(i, _):
        v = idxb_v[pl.ds(i * LANES, LANES)]
        rel = v - mlo
        inr = (rel >= 0) & (rel < MPT)
        relc = jnp.where(inr, rel, MPT)
        plsc.addupdate_scatter(scale_v, [relc], ones)
        return 0

    lax.fori_loop(0, L // LANES, hist_body, 0)

    def recip_body(i, _):
        sl = pl.ds(i * LANES, LANES)
        scale_v[sl] = 1.0 / scale_v[sl]
        return 0

    lax.fori_loop(0, (MPT + LANES) // LANES, recip_body, 0)

    def scan_body(i, ns):
        v = idxb_v[pl.ds(i * LANES, LANES)]
        rel = v - mlo
        inr = (rel >= 0) & (rel < MPT)
        packed = (rel << CBITS) | (i * LANES + iota)
        plsc.store_compressed(stage_v.at[pl.ds(ns, LANES)], packed, mask=inr)
        cnt = plsc.all_reduce_population_count(inr)
        return ns + cnt[0]

    ns = lax.fori_loop(0, L // LANES, scan_body, 0)
    nsv = (ns + LANES - 1) // LANES

    def fire_scatter(slot):
        pltpu.async_copy(sbuf_v.at[slot], out_hbm.at[pidx_v.at[slot]],
                         osems[slot])

    def wait_scatter(slot):
        pltpu.make_async_copy(sbuf_v.at[slot], out_hbm.at[pidx_v.at[slot]],
                              osems[slot]).wait()

    def do_sub(r, half, fs):
        @pl.when(r + 1 < NSUB)
        def _():
            start_slab(r + 1, (half + 1) % 2)

        wait_slab(r, half)

        def sel_body(i, m):
            w = stage_v[pl.ds(i * LANES, LANES)]
            valid = (i * LANES + iota) < ns
            hit = valid & ((w >> (CBITS + 5)) == r)
            plsc.store_compressed(work_v.at[pl.ds(m, LANES)], w, mask=hit)
            cnt = plsc.all_reduce_population_count(hit)
            return m + cnt[0]

        m_r = lax.fori_loop(0, nsv, sel_body, 0)

        @pl.when(m_r > 0)
        def _():
            first = plsc.load_gather(work_v, [jnp.zeros((LANES,), jnp.int32)])
            padcnt = (-m_r) & (LANES - 1)
            work_v[pl.ds(m_r, LANES)] = jnp.where(iota < padcnt, first,
                                                  jnp.int32(0))

        mchunks = (m_r + LANES - 1) // LANES

        def build_fire(q, slot):
            w = work_v[pl.ds(q * LANES, LANES)]
            pidx_v[slot, pl.ds(0, LANES)] = (w & (L - 1)) + b * L

            def entry_body(j, _):
                u = plsc.load_gather(work_v, [jnp.full((LANES,), q * LANES + j,
                                                       jnp.int32)])
                rel_u = u >> CBITS
                su = plsc.load_gather(scale_v, [rel_u])
                off_u = rel_u - r * SUB
                for k in range(D // LANES):
                    col = k * LANES + iota
                    vals = plsc.load_gather(slab_v.at[half], [off_u, col])
                    sbuf_v[slot, j, pl.ds(k * LANES, LANES)] = vals * su
                return 0

            lax.fori_loop(0, LANES, entry_body, 0)
            fire_scatter(slot)

        def block_pair(qq, fs):
            f0, f1 = fs

            @pl.when(f0 == 1)
            def _():
                wait_scatter(0)

            build_fire(qq * 2, 0)
            g = (qq * 2 + 1) < mchunks

            @pl.when(g & (f1 == 1))
            def _():
                wait_scatter(1)

            @pl.when(g)
            def _():
                build_fire(qq * 2 + 1, 1)

            return (jnp.int32(1), jnp.where(g, jnp.int32(1), f1))

        return lax.fori_loop(0, (mchunks + 1) // 2, block_pair, fs)

    def sub_body(rr, fs):
        fs = do_sub(rr * 2, 0, fs)
        fs = do_sub(rr * 2 + 1, 1, fs)
        return fs

    f0, f1 = lax.fori_loop(0, NSUB // 2, sub_body,
                           (jnp.int32(0), jnp.int32(0)))

    @pl.when(f0 == 1)
    def _():
        wait_scatter(0)

    @pl.when(f1 == 1)
    def _():
        wait_scatter(1)


_unmerge_call = functools.partial(
    pl.kernel,
    out_type=jax.ShapeDtypeStruct((B * L, D), jnp.float32),
    mesh=plsc.VectorSubcoreMesh(core_axis_name="c", subcore_axis_name="s",
                                num_cores=NC, num_subcores=NS),
    scratch_types=[
        pltpu.VMEM((L,), jnp.int32),
        pltpu.VMEM((MPT + LANES,), jnp.float32),
        pltpu.VMEM((CAP,), jnp.int32),
        pltpu.VMEM((CAP,), jnp.int32),
        pltpu.VMEM((2, SUB, D), jnp.float32),
        pltpu.VMEM((2, LANES, D), jnp.float32),
        pltpu.VMEM((2, LANES), jnp.int32),
        pltpu.SemaphoreType.DMA,
        pltpu.SemaphoreType.DMA,
        pltpu.SemaphoreType.DMA,
        pltpu.SemaphoreType.DMA,
    ],
    compiler_params=pltpu.CompilerParams(needs_layout_passes=False),
)(_unmerge_body)


@jax.jit
def kernel(merged_feats, source_maps):
    source_idx = source_maps[0]
    for i in range(1, source_maps.shape[0]):
        source_idx = jnp.take_along_axis(source_maps[i], source_idx, axis=1)
    assert merged_feats.shape == (B, M, D)
    assert source_idx.shape == (B, L)
    out = _unmerge_call(merged_feats.reshape(B * M, D),
                        source_idx.astype(jnp.int32))
    return out.reshape(B, L, D)


# speedup vs baseline: 1.0140x; 1.0140x over previous
import functools

import jax
import jax.numpy as jnp
from jax import lax
from jax.experimental import pallas as pl
from jax.experimental.pallas import tpu as pltpu
from jax.experimental.pallas import tpu_sc as plsc

B, M, L, D = 4, 4096, 8192, 1024
NC, NS, LANES = 2, 16, 16
NW = NC * NS
TPB = NW // B
MPT = M // TPB
SUB = 32
NSUB = MPT // SUB
CAP = L + LANES
CBITS = 13
ROWB = SUB * D * 4
SCATB = LANES * D * 4


def _unmerge_body(merged_hbm, idx_hbm, out_hbm,
                  idxb_v, scale_v, stage_v, work_v, slab_v, sbuf_v, pidx_v,
                  ssem0, ssem1, osem0, osem1):
    ssems = (ssem0, ssem1)
    osems = (osem0, osem1)
    cid = lax.axis_index("c")
    sid = lax.axis_index("s")
    wid = sid * NC + cid
    b = wid // TPB
    mlo = (wid % TPB) * MPT
    sbase = wid * MPT

    iota = lax.iota(jnp.int32, LANES)

    pltpu.sync_copy(idx_hbm.at[b], idxb_v)

    def start_slab(r, slot):
        pltpu.async_copy(merged_hbm.at[pl.ds(sbase + r * SUB, SUB)],
                         slab_v.at[slot], ssems[slot])

    def wait_slab(r, slot):
        pltpu.make_async_copy(merged_hbm.at[pl.ds(sbase + r * SUB, SUB)],
                              slab_v.at[slot], ssems[slot]).wait()

    start_slab(0, 0)

    zeros = jnp.zeros((LANES,), jnp.float32)

    def zero_body(i, _):
        scale_v[pl.ds(i * LANES, LANES)] = zeros
        return 0

    lax.fori_loop(0, (MPT + LANES) // LANES, zero_body, 0)

    ones = jnp.ones((LANES,), jnp.float32)

    def hist_body(i, _):
        v = idxb_v[pl.ds(i * LANES, LANES)]
        rel = v - mlo
        inr = (rel >= 0) & (rel < MPT)
        relc = jnp.where(inr, rel, MPT)
        plsc.addupdate_scatter(scale_v, [relc], ones)
        return 0

    lax.fori_loop(0, L // LANES, hist_body, 0)

    def recip_body(i, _):
        sl = pl.ds(i * LANES, LANES)
        scale_v[sl] = 1.0 / scale_v[sl]
        return 0

    lax.fori_loop(0, (MPT + LANES) // LANES, recip_body, 0)

    def scan_body(i, ns):
        v = idxb_v[pl.ds(i * LANES, LANES)]
        rel = v - mlo
        inr = (rel >= 0) & (rel < MPT)
        packed = (rel << CBITS) | (i * LANES + iota)
        plsc.store_compressed(stage_v.at[pl.ds(ns, LANES)], packed, mask=inr)
        cnt = plsc.all_reduce_population_count(inr)
        return ns + cnt[0]

    ns = lax.fori_loop(0, L // LANES, scan_body, 0)
    nsv = (ns + LANES - 1) // LANES

    def fire_scatter(slot):
        pltpu.async_copy(sbuf_v.at[slot], out_hbm.at[pidx_v.at[slot]],
                         osems[slot])

    def wait_scatter(slot):
        pltpu.make_async_copy(sbuf_v.at[slot], out_hbm.at[pidx_v.at[slot]],
                              osems[slot]).wait()

    def do_sub(r, half, fs):
        @pl.when(r + 1 < NSUB)
        def _():
            start_slab(r + 1, (half + 1) % 2)

        wait_slab(r, half)

        def sel_body(i, m):
            w = stage_v[pl.ds(i * LANES, LANES)]
            valid = (i * LANES + iota) < ns
            hit = valid & ((w >> (CBITS + 5)) == r)
            plsc.store_compressed(work_v.at[pl.ds(m, LANES)], w, mask=hit)
            cnt = plsc.all_reduce_population_count(hit)
            return m + cnt[0]

        m_r = lax.fori_loop(0, nsv, sel_body, 0)

        @pl.when(m_r > 0)
        def _():
            first = plsc.load_gather(work_v, [jnp.zeros((LANES,), jnp.int32)])
            padcnt = (-m_r) & (LANES - 1)
            work_v[pl.ds(m_r, LANES)] = jnp.where(iota < padcnt, first,
                                                  jnp.int32(0))

        mchunks = (m_r + LANES - 1) // LANES

        def build_fire(q, slot):
            w = work_v[pl.ds(q * LANES, LANES)]
            pidx_v[slot, pl.ds(0, LANES)] = (w & (L - 1)) + b * L

            def entry_body(j, _):
                u = plsc.load_gather(work_v, [jnp.full((LANES,), q * LANES + j,
                                                       jnp.int32)])
                rel_u = u >> CBITS
                su = plsc.load_gather(scale_v, [rel_u])
                off_s = rel_u[0] - r * SUB
                for k in range(D // LANES):
                    sl = pl.ds(k * LANES, LANES)
                    sbuf_v[slot, j, sl] = slab_v[half, off_s, sl] * su
                return 0

            lax.fori_loop(0, LANES, entry_body, 0)
            fire_scatter(slot)

        def block_pair(qq, fs):
            f0, f1 = fs

            @pl.when(f0 == 1)
            def _():
                wait_scatter(0)

            build_fire(qq * 2, 0)
            g = (qq * 2 + 1) < mchunks

            @pl.when(g & (f1 == 1))
            def _():
                wait_scatter(1)

            @pl.when(g)
            def _():
                build_fire(qq * 2 + 1, 1)

            return (jnp.int32(1), jnp.where(g, jnp.int32(1), f1))

        return lax.fori_loop(0, (mchunks + 1) // 2, block_pair, fs)

    def sub_body(rr, fs):
        fs = do_sub(rr * 2, 0, fs)
        fs = do_sub(rr * 2 + 1, 1, fs)
        return fs

    f0, f1 = lax.fori_loop(0, NSUB // 2, sub_body,
                           (jnp.int32(0), jnp.int32(0)))

    @pl.when(f0 == 1)
    def _():
        wait_scatter(0)

    @pl.when(f1 == 1)
    def _():
        wait_scatter(1)


_unmerge_call = functools.partial(
    pl.kernel,
    out_type=jax.ShapeDtypeStruct((B * L, D), jnp.float32),
    mesh=plsc.VectorSubcoreMesh(core_axis_name="c", subcore_axis_name="s",
                                num_cores=NC, num_subcores=NS),
    scratch_types=[
        pltpu.VMEM((L,), jnp.int32),
        pltpu.VMEM((MPT + LANES,), jnp.float32),
        pltpu.VMEM((CAP,), jnp.int32),
        pltpu.VMEM((CAP,), jnp.int32),
        pltpu.VMEM((2, SUB, D), jnp.float32),
        pltpu.VMEM((2, LANES, D), jnp.float32),
        pltpu.VMEM((2, LANES), jnp.int32),
        pltpu.SemaphoreType.DMA,
        pltpu.SemaphoreType.DMA,
        pltpu.SemaphoreType.DMA,
        pltpu.SemaphoreType.DMA,
    ],
    compiler_params=pltpu.CompilerParams(needs_layout_passes=False),
)(_unmerge_body)


@jax.jit
def kernel(merged_feats, source_maps):
    source_idx = source_maps[0]
    for i in range(1, source_maps.shape[0]):
        source_idx = jnp.take_along_axis(source_maps[i], source_idx, axis=1)
    assert merged_feats.shape == (B, M, D)
    assert source_idx.shape == (B, L)
    out = _unmerge_call(merged_feats.reshape(B * M, D),
                        source_idx.astype(jnp.int32))
    return out.reshape(B, L, D)


# speedup vs baseline: 3.6465x; 3.5962x over previous
import functools

import jax
import jax.numpy as jnp
from jax import lax
from jax.experimental import pallas as pl
from jax.experimental.pallas import tpu as pltpu
from jax.experimental.pallas import tpu_sc as plsc

B, M, L, D = 4, 4096, 8192, 1024
NC, NS, LANES = 2, 16, 16
NW = NC * NS
TPB = NW // B
RPT = L // TPB
CHUNK = 16
NCH = RPT // CHUNK
NBUF = 4
PF = 3
NGRP = NCH // NBUF


def _unmerge_body(merged_hbm, idx_hbm, out_hbm,
                  idxb_v, scale_v, gidx_v, scl_v, rows_v,
                  *sems):
    gsems, osems = sems[:NBUF], sems[NBUF:]
    cid = lax.axis_index("c")
    sid = lax.axis_index("s")
    wid = sid * NC + cid
    b = wid // TPB
    lbase = (wid % TPB) * RPT

    pltpu.sync_copy(idx_hbm.at[b], idxb_v)

    obase = wid * RPT

    def gidx_body(c, _):
        for h in range(CHUNK // LANES):
            v = idxb_v[pl.ds(lbase + c * CHUNK + h * LANES, LANES)]
            gidx_v[c, pl.ds(h * LANES, LANES)] = v + b * M
        return 0

    lax.fori_loop(0, NCH, gidx_body, 0)

    def start_gather(c, buf):
        pltpu.async_copy(merged_hbm.at[gidx_v.at[c]], rows_v.at[buf],
                         gsems[buf])

    for c0 in range(PF):
        start_gather(c0, c0)

    zeros = jnp.zeros((LANES,), jnp.float32)

    def zero_body(i, _):
        scale_v[pl.ds(i * LANES, LANES)] = zeros
        return 0

    lax.fori_loop(0, M // LANES, zero_body, 0)

    ones = jnp.ones((LANES,), jnp.float32)

    def hist_body(i, _):
        v = idxb_v[pl.ds(i * LANES, LANES)]
        plsc.addupdate_scatter(scale_v, [v], ones)
        return 0

    lax.fori_loop(0, L // LANES, hist_body, 0)

    def recip_body(i, _):
        sl = pl.ds(i * LANES, LANES)
        scale_v[sl] = 1.0 / scale_v[sl]
        return 0

    lax.fori_loop(0, M // LANES, recip_body, 0)

    def scl_body(j, _):
        v = idxb_v[pl.ds(lbase + j * LANES, LANES)]
        scl_v[pl.ds(j * LANES, LANES)] = plsc.load_gather(scale_v, [v])
        return 0

    lax.fori_loop(0, RPT // LANES, scl_body, 0)

    def wait_gather(c, buf):
        pltpu.make_async_copy(merged_hbm.at[gidx_v.at[c]], rows_v.at[buf],
                              gsems[buf]).wait()

    def start_out(c, buf):
        pltpu.async_copy(rows_v.at[buf],
                         out_hbm.at[pl.ds(obase + c * CHUNK, CHUNK)],
                         osems[buf])

    def wait_out(c, buf):
        pltpu.make_async_copy(rows_v.at[buf],
                              out_hbm.at[pl.ds(obase + c * CHUNK, CHUNK)],
                              osems[buf]).wait()

    def scale_chunk(c, buf):
        def row_body(j, _):
            idx16 = jnp.full((LANES,), c * CHUNK + j, jnp.int32)
            s = plsc.load_gather(scl_v, [idx16])
            for k in range(D // LANES):
                sl = pl.ds(k * LANES, LANES)
                rows_v[buf, j, sl] = rows_v[buf, j, sl] * s
            return 0

        lax.fori_loop(0, CHUNK, row_body, 0)

    def slot(c, buf, is_static):
        wait_gather(c, buf)
        scale_chunk(c, buf)
        start_out(c, buf)
        nc = c + PF
        nb = (buf + PF) % NBUF
        if is_static:
            if nc >= NBUF:
                wait_out(nc - NBUF, nb)
            if nc < NCH:
                start_gather(nc, nb)
        else:
            @pl.when(nc >= NBUF)
            def _():
                wait_out(nc - NBUF, nb)

            @pl.when(nc < NCH)
            def _():
                start_gather(nc, nb)

    def group_body(g, _):
        for buf in range(NBUF):
            slot(g * NBUF + buf, buf, False)
        return 0

    lax.fori_loop(0, NGRP, group_body, 0)
    for c0 in range(NGRP * NBUF, NCH):
        slot(c0, c0 % NBUF, True)

    for c0 in range(NCH - (NBUF - PF), NCH):
        wait_out(c0, c0 % NBUF)


_unmerge_call = functools.partial(
    pl.kernel,
    out_type=jax.ShapeDtypeStruct((B * L, D), jnp.float32),
    mesh=plsc.VectorSubcoreMesh(core_axis_name="c", subcore_axis_name="s",
                                num_cores=NC, num_subcores=NS),
    scratch_types=[
        pltpu.VMEM((L,), jnp.int32),
        pltpu.VMEM((M,), jnp.float32),
        pltpu.VMEM((NCH, CHUNK), jnp.int32),
        pltpu.VMEM((RPT,), jnp.float32),
        pltpu.VMEM((NBUF, CHUNK, D), jnp.float32),
    ] + [pltpu.SemaphoreType.DMA] * (2 * NBUF),
    compiler_params=pltpu.CompilerParams(needs_layout_passes=False),
)(_unmerge_body)


@jax.jit
def kernel(merged_feats, source_maps):
    source_idx = source_maps[0]
    for i in range(1, source_maps.shape[0]):
        source_idx = jnp.take_along_axis(source_maps[i], source_idx, axis=1)
    assert merged_feats.shape == (B, M, D)
    assert source_idx.shape == (B, L)
    out = _unmerge_call(merged_feats.reshape(B * M, D),
                        source_idx.astype(jnp.int32))
    return out.reshape(B, L, D)
